# baseline (speedup 1.0000x reference)
import functools

import jax
import jax.numpy as jnp
from jax import lax
from jax.experimental import pallas as pl
from jax.experimental.pallas import tpu as pltpu
from jax.experimental.pallas import tpu_sc as plsc

N_NODES = 512
KSIZE = 16
N_CHAN = 8
N_STEPS = 2
NUM_WORKERS = 32
ROWS_PER_W = N_NODES // NUM_WORKERS
N_GROUPS = N_NODES // 4


def _worker_base():
    wid = lax.axis_index("s") * 2 + lax.axis_index("c")
    return wid * ROWS_PER_W


def _build_tables(mt_v, x_v, tab_v, nib_v):
    def body(g, carry):
        ms = [jnp.minimum(mt_v[4 * g + k, :], 1) for k in range(4)]
        nib = ms[0] + (ms[1] << 1) + (ms[2] << 2) + (ms[3] << 3)
        nib_v[g, :] = (g << 8) + (nib << 4)
        xs = [x_v[4 * g + k, :] for k in range(4)]
        tab_v[pl.ds(g * 256, 16)] = jnp.zeros((KSIZE,), jnp.float32)
        vals = {}
        for s in range(1, 16):
            k = (s & -s).bit_length() - 1
            prev = s ^ (1 << k)
            vals[s] = xs[k] if prev == 0 else vals[prev] + xs[k]
            tab_v[pl.ds(g * 256 + s * 16, 16)] = vals[s]
        return carry

    lax.fori_loop(0, N_GROUPS, body, 0)


def _gather_accumulate(tab_v, nib_v, tr_v):
    def body(g, accs):
        base = nib_v[g, :]
        return tuple(accs[d] + plsc.load_gather(tab_v, [base + d])
                     for d in range(KSIZE))

    zero = jnp.zeros((KSIZE,), jnp.float32)
    accs = lax.fori_loop(0, N_GROUPS, body,
                         tuple(zero for _ in range(KSIZE)))
    for d in range(KSIZE):
        tr_v[pl.ds(d * 16, 16)] = accs[d]
    lanes16 = lax.iota(jnp.int32, 16) * 16
    return [plsc.load_gather(tr_v, [lanes16 + r]) for r in range(ROWS_PER_W)]


_SCRATCH_COMMON = [
    pltpu.VMEM((N_NODES, ROWS_PER_W), jnp.int32),
    pltpu.VMEM((N_NODES, KSIZE), jnp.float32),
    pltpu.VMEM((N_GROUPS * 256,), jnp.float32),
    pltpu.VMEM((N_GROUPS, 16), jnp.int32),
    pltpu.VMEM((256,), jnp.float32),
    pltpu.SemaphoreType.DMA,
    pltpu.SemaphoreType.DMA,
]


@functools.cache
def _build_calls():
    mesh = plsc.VectorSubcoreMesh(core_axis_name="c", subcore_axis_name="s")

    @functools.partial(
        pl.kernel,
        out_type=jax.ShapeDtypeStruct((N_NODES, KSIZE), jnp.float32),
        mesh=mesh,
        compiler_params=pltpu.CompilerParams(use_tc_tiling_on_sc=False, needs_layout_passes=False),
        scratch_types=_SCRATCH_COMMON + [
            pltpu.VMEM((ROWS_PER_W, KSIZE), jnp.float32),
        ],
    )
    def aggregate(mt_hbm, x_hbm, out_hbm,
                  mt_v, x_v, tab_v, nib_v, tr_v, sem_a, sem_b, o_v):
        base = _worker_base()
        wid = base // ROWS_PER_W
        cp_a = pltpu.async_copy(mt_hbm.at[wid], mt_v, sem_a)
        cp_b = pltpu.async_copy(x_hbm, x_v, sem_b)
        cp_a.wait()
        cp_b.wait()
        _build_tables(mt_v, x_v, tab_v, nib_v)
        rows = _gather_accumulate(tab_v, nib_v, tr_v)
        for r in range(ROWS_PER_W):
            o_v[r, :] = rows[r]
        pltpu.sync_copy(o_v, out_hbm.at[pl.ds(base, ROWS_PER_W), :])

    @functools.partial(
        pl.kernel,
        out_type=jax.ShapeDtypeStruct((N_CHAN * N_NODES, KSIZE), jnp.float32),
        mesh=mesh,
        compiler_params=pltpu.CompilerParams(use_tc_tiling_on_sc=False, needs_layout_passes=False),
        scratch_types=_SCRATCH_COMMON + [
            pltpu.VMEM((ROWS_PER_W, KSIZE), jnp.float32),
            pltpu.VMEM((N_CHAN * N_STEPS, KSIZE), jnp.float32),
            pltpu.VMEM((N_CHAN, ROWS_PER_W, KSIZE), jnp.float32),
        ],
    )
    def aggregate_combine(mt_hbm, p_hbm, l_hbm, k_hbm, out_hbm,
                          mt_v, p_v, tab_v, nib_v, tr_v, sem_a, sem_b,
                          l_v, k_v, o_v):
        base = _worker_base()
        wid = base // ROWS_PER_W
        cp_a = pltpu.async_copy(mt_hbm.at[wid], mt_v, sem_a)
        cp_b = pltpu.async_copy(p_hbm, p_v, sem_b)
        pltpu.sync_copy(l_hbm.at[pl.ds(base, ROWS_PER_W), :], l_v)
        pltpu.sync_copy(k_hbm, k_v)
        cp_a.wait()
        cp_b.wait()
        _build_tables(mt_v, p_v, tab_v, nib_v)
        qs = _gather_accumulate(tab_v, nib_v, tr_v)
        for r in range(ROWS_PER_W):
            q = qs[r]
            p_i = p_v[base + r, :]
            l_i = l_v[r, :]
            for c in range(N_CHAN):
                k0 = k_v[2 * c, :]
                k1 = k_v[2 * c + 1, :]
                o_v[c, r, :] = l_i + p_i * (k0 + k1) + q * (k0 * k1)
        for c in range(N_CHAN):
            pltpu.sync_copy(
                o_v.at[c],
                out_hbm.at[pl.ds(c * N_NODES + base, ROWS_PER_W), :])

    return aggregate, aggregate_combine


def kernel(labelsList, ligand_structure, kernels):
    aggregate, aggregate_combine = _build_calls()
    mtb = ligand_structure.reshape(NUM_WORKERS, ROWS_PER_W, N_NODES)
    mtb = mtb.transpose(0, 2, 1)
    p = aggregate(mtb, labelsList)
    flat_k = kernels.reshape(N_CHAN * N_STEPS, KSIZE)
    out = aggregate_combine(mtb, p, labelsList, flat_k)
    return out.reshape(N_CHAN, N_NODES, KSIZE)


# speedup vs baseline: 1.0537x; 1.0537x over previous
import functools

import jax
import jax.numpy as jnp
from jax import lax
from jax.experimental import pallas as pl
from jax.experimental.pallas import tpu as pltpu
from jax.experimental.pallas import tpu_sc as plsc

N_NODES = 512
KSIZE = 16
N_CHAN = 8
N_STEPS = 2
NUM_WORKERS = 32
ROWS_PER_W = N_NODES // NUM_WORKERS
N_GROUPS = N_NODES // 4


def _worker_base():
    wid = lax.axis_index("s") * 2 + lax.axis_index("c")
    return wid * ROWS_PER_W


def _build_tables(m_v, x_v, tab_v, nib_v, tr_i):
    lanes16 = lax.iota(jnp.int32, 16) * 16

    def body(th, carry):
        for u in range(2):
            t = 2 * th + u
            off = u * 256
            for r in range(ROWS_PER_W):
                tr_i[pl.ds(off + r * 16, 16)] = m_v[r, pl.ds(t * 16, 16)]
            cols = [plsc.load_gather(tr_i, [lanes16 + (off + c)])
                    for c in range(16)]
            for k in range(4):
                g = 4 * t + k
                ms = [jnp.minimum(cols[4 * k + i], 1) for i in range(4)]
                nib = ms[0] + (ms[1] << 1) + (ms[2] << 2) + (ms[3] << 3)
                nib_v[g, :] = (g << 8) + (nib << 4)
                xs = [x_v[4 * g + i, :] for i in range(4)]
                tab_v[pl.ds(g * 256, 16)] = jnp.zeros((KSIZE,), jnp.float32)
                vals = {}
                for s in range(1, 16):
                    kk = (s & -s).bit_length() - 1
                    prev = s ^ (1 << kk)
                    vals[s] = xs[kk] if prev == 0 else vals[prev] + xs[kk]
                    tab_v[pl.ds(g * 256 + s * 16, 16)] = vals[s]
        return carry

    lax.fori_loop(0, N_GROUPS // 8, body, 0)


def _gather_accumulate(tab_v, nib_v, tr_v):
    def body(g, accs):
        base = nib_v[g, :]
        return tuple(accs[d] + plsc.load_gather(tab_v, [base + d])
                     for d in range(KSIZE))

    zero = jnp.zeros((KSIZE,), jnp.float32)
    accs = lax.fori_loop(0, N_GROUPS, body,
                         tuple(zero for _ in range(KSIZE)))
    for d in range(KSIZE):
        tr_v[pl.ds(d * 16, 16)] = accs[d]
    lanes16 = lax.iota(jnp.int32, 16) * 16
    return [plsc.load_gather(tr_v, [lanes16 + r]) for r in range(ROWS_PER_W)]


_SCRATCH_COMMON = [
    pltpu.VMEM((ROWS_PER_W, N_NODES), jnp.int32),
    pltpu.VMEM((N_NODES, KSIZE), jnp.float32),
    pltpu.VMEM((N_GROUPS * 256,), jnp.float32),
    pltpu.VMEM((N_GROUPS, 16), jnp.int32),
    pltpu.VMEM((256,), jnp.float32),
    pltpu.VMEM((512,), jnp.int32),
    pltpu.SemaphoreType.DMA,
    pltpu.SemaphoreType.DMA,
]


@functools.cache
def _build_calls():
    mesh = plsc.VectorSubcoreMesh(core_axis_name="c", subcore_axis_name="s")

    @functools.partial(
        pl.kernel,
        out_type=jax.ShapeDtypeStruct((N_NODES, KSIZE), jnp.float32),
        mesh=mesh,
        compiler_params=pltpu.CompilerParams(use_tc_tiling_on_sc=False, needs_layout_passes=False),
        scratch_types=_SCRATCH_COMMON + [
            pltpu.VMEM((ROWS_PER_W, KSIZE), jnp.float32),
        ],
    )
    def aggregate(m_hbm, x_hbm, out_hbm,
                  m_v, x_v, tab_v, nib_v, tr_v, tr_i, sem_a, sem_b, o_v):
        base = _worker_base()
        cp_a = pltpu.async_copy(m_hbm.at[pl.ds(base, ROWS_PER_W), :], m_v,
                                sem_a)
        cp_b = pltpu.async_copy(x_hbm, x_v, sem_b)
        cp_a.wait()
        cp_b.wait()
        _build_tables(m_v, x_v, tab_v, nib_v, tr_i)
        rows = _gather_accumulate(tab_v, nib_v, tr_v)
        for r in range(ROWS_PER_W):
            o_v[r, :] = rows[r]
        pltpu.sync_copy(o_v, out_hbm.at[pl.ds(base, ROWS_PER_W), :])

    @functools.partial(
        pl.kernel,
        out_type=jax.ShapeDtypeStruct((N_CHAN * N_NODES, KSIZE), jnp.float32),
        mesh=mesh,
        compiler_params=pltpu.CompilerParams(use_tc_tiling_on_sc=False, needs_layout_passes=False),
        scratch_types=_SCRATCH_COMMON + [
            pltpu.VMEM((ROWS_PER_W, KSIZE), jnp.float32),
            pltpu.VMEM((N_CHAN * N_STEPS, KSIZE), jnp.float32),
            pltpu.VMEM((N_CHAN, ROWS_PER_W, KSIZE), jnp.float32),
        ],
    )
    def aggregate_combine(m_hbm, p_hbm, l_hbm, k_hbm, out_hbm,
                          m_v, p_v, tab_v, nib_v, tr_v, tr_i, sem_a, sem_b,
                          l_v, k_v, o_v):
        base = _worker_base()
        cp_a = pltpu.async_copy(m_hbm.at[pl.ds(base, ROWS_PER_W), :], m_v,
                                sem_a)
        cp_b = pltpu.async_copy(p_hbm, p_v, sem_b)
        pltpu.sync_copy(l_hbm.at[pl.ds(base, ROWS_PER_W), :], l_v)
        pltpu.sync_copy(k_hbm, k_v)
        cp_a.wait()
        cp_b.wait()
        _build_tables(m_v, p_v, tab_v, nib_v, tr_i)
        qs = _gather_accumulate(tab_v, nib_v, tr_v)
        for r in range(ROWS_PER_W):
            q = qs[r]
            p_i = p_v[base + r, :]
            l_i = l_v[r, :]
            for c in range(N_CHAN):
                k0 = k_v[2 * c, :]
                k1 = k_v[2 * c + 1, :]
                o_v[c, r, :] = l_i + p_i * (k0 + k1) + q * (k0 * k1)
        for c in range(N_CHAN):
            pltpu.sync_copy(
                o_v.at[c],
                out_hbm.at[pl.ds(c * N_NODES + base, ROWS_PER_W), :])

    return aggregate, aggregate_combine


def kernel(labelsList, ligand_structure, kernels):
    aggregate, aggregate_combine = _build_calls()
    p = aggregate(ligand_structure, labelsList)
    flat_k = kernels.reshape(N_CHAN * N_STEPS, KSIZE)
    out = aggregate_combine(ligand_structure, p, labelsList, flat_k)
    return out.reshape(N_CHAN, N_NODES, KSIZE)


# speedup vs baseline: 1.3399x; 1.2716x over previous
import functools

import jax
import jax.numpy as jnp
from jax import lax
from jax.experimental import pallas as pl
from jax.experimental.pallas import tpu as pltpu
from jax.experimental.pallas import tpu_sc as plsc

N_NODES = 512
KSIZE = 16
N_CHAN = 8
N_STEPS = 2
NUM_WORKERS = 32
ROWS_PER_W = N_NODES // NUM_WORKERS
N_GROUPS = N_NODES // 4


def _worker_base():
    wid = lax.axis_index("s") * 2 + lax.axis_index("c")
    return wid * ROWS_PER_W


def _build_tables(x_v, tab_v):
    def body(th, carry):
        for u in range(2):
            g = 2 * th + u
            xs = [x_v[4 * g + i, :] for i in range(4)]
            tab_v[pl.ds(g * 256, 16)] = jnp.zeros((KSIZE,), jnp.float32)
            vals = {}
            for s in range(1, 16):
                kk = (s & -s).bit_length() - 1
                prev = s ^ (1 << kk)
                vals[s] = xs[kk] if prev == 0 else vals[prev] + xs[kk]
                tab_v[pl.ds(g * 256 + s * 16, 16)] = vals[s]
        return carry

    lax.fori_loop(0, N_GROUPS // 2, body, 0)


def _masked_rowsums(m_v, tab_v):
    iota = lax.iota(jnp.int32, 16)
    sh4 = iota & 3
    fold1 = iota ^ 1
    fold2 = iota ^ 2
    rows = []

    for r0 in range(0, ROWS_PER_W, 4):

        def body(t, accs, r0=r0):
            out = []
            for ri in range(4):
                r = r0 + ri
                mrow = jnp.minimum(m_v[r, pl.ds(t * 16, 16)], 1)
                sh = mrow << sh4
                s1 = sh + sh.at[fold1].get(mode="promise_in_bounds")
                nib = s1 + s1.at[fold2].get(mode="promise_in_bounds")
                acc = accs[ri]
                for k in range(4):
                    nb = nib.at[jnp.full((16,), 4 * k, jnp.int32)].get(
                        mode="promise_in_bounds")
                    vidx = (nb << 4) + ((4 * t + k) << 8) + iota
                    acc = acc + plsc.load_gather(tab_v, [vidx])
                out.append(acc)
            return tuple(out)

        zero = jnp.zeros((KSIZE,), jnp.float32)
        accs = lax.fori_loop(0, N_NODES // 16, body,
                             tuple(zero for _ in range(4)))
        rows.extend(accs)
    return rows


_SCRATCH_COMMON = [
    pltpu.VMEM((ROWS_PER_W, N_NODES), jnp.int32),
    pltpu.VMEM((N_NODES, KSIZE), jnp.float32),
    pltpu.VMEM((N_GROUPS * 256,), jnp.float32),
    pltpu.SemaphoreType.DMA,
    pltpu.SemaphoreType.DMA,
]


@functools.cache
def _build_calls():
    mesh = plsc.VectorSubcoreMesh(core_axis_name="c", subcore_axis_name="s")

    @functools.partial(
        pl.kernel,
        out_type=jax.ShapeDtypeStruct((N_NODES, KSIZE), jnp.float32),
        mesh=mesh,
        compiler_params=pltpu.CompilerParams(use_tc_tiling_on_sc=False, needs_layout_passes=False),
        scratch_types=_SCRATCH_COMMON + [
            pltpu.VMEM((ROWS_PER_W, KSIZE), jnp.float32),
        ],
    )
    def aggregate(m_hbm, x_hbm, out_hbm,
                  m_v, x_v, tab_v, sem_a, sem_b, o_v):
        base = _worker_base()
        cp_a = pltpu.async_copy(m_hbm.at[pl.ds(base, ROWS_PER_W), :], m_v,
                                sem_a)
        cp_b = pltpu.async_copy(x_hbm, x_v, sem_b)
        cp_b.wait()
        _build_tables(x_v, tab_v)
        cp_a.wait()
        rows = _masked_rowsums(m_v, tab_v)
        for r in range(ROWS_PER_W):
            o_v[r, :] = rows[r]
        pltpu.sync_copy(o_v, out_hbm.at[pl.ds(base, ROWS_PER_W), :])

    @functools.partial(
        pl.kernel,
        out_type=jax.ShapeDtypeStruct((N_CHAN * N_NODES, KSIZE), jnp.float32),
        mesh=mesh,
        compiler_params=pltpu.CompilerParams(use_tc_tiling_on_sc=False, needs_layout_passes=False),
        scratch_types=_SCRATCH_COMMON + [
            pltpu.VMEM((ROWS_PER_W, KSIZE), jnp.float32),
            pltpu.VMEM((N_CHAN * N_STEPS, KSIZE), jnp.float32),
            pltpu.VMEM((N_CHAN, ROWS_PER_W, KSIZE), jnp.float32),
        ],
    )
    def aggregate_combine(m_hbm, p_hbm, l_hbm, k_hbm, out_hbm,
                          m_v, p_v, tab_v, sem_a, sem_b,
                          l_v, k_v, o_v):
        base = _worker_base()
        cp_a = pltpu.async_copy(m_hbm.at[pl.ds(base, ROWS_PER_W), :], m_v,
                                sem_a)
        cp_b = pltpu.async_copy(p_hbm, p_v, sem_b)
        pltpu.sync_copy(l_hbm.at[pl.ds(base, ROWS_PER_W), :], l_v)
        pltpu.sync_copy(k_hbm, k_v)
        cp_b.wait()
        _build_tables(p_v, tab_v)
        cp_a.wait()
        qs = _masked_rowsums(m_v, tab_v)
        for r in range(ROWS_PER_W):
            q = qs[r]
            p_i = p_v[base + r, :]
            l_i = l_v[r, :]
            for c in range(N_CHAN):
                k0 = k_v[2 * c, :]
                k1 = k_v[2 * c + 1, :]
                o_v[c, r, :] = l_i + p_i * (k0 + k1) + q * (k0 * k1)
        for c in range(N_CHAN):
            pltpu.sync_copy(
                o_v.at[c],
                out_hbm.at[pl.ds(c * N_NODES + base, ROWS_PER_W), :])

    return aggregate, aggregate_combine


def kernel(labelsList, ligand_structure, kernels):
    aggregate, aggregate_combine = _build_calls()
    p = aggregate(ligand_structure, labelsList)
    flat_k = kernels.reshape(N_CHAN * N_STEPS, KSIZE)
    out = aggregate_combine(ligand_structure, p, labelsList, flat_k)
    return out.reshape(N_CHAN, N_NODES, KSIZE)


# speedup vs baseline: 1.3856x; 1.0341x over previous
import functools

import jax
import jax.numpy as jnp
from jax import lax
from jax.experimental import pallas as pl
from jax.experimental.pallas import tpu as pltpu
from jax.experimental.pallas import tpu_sc as plsc

N_NODES = 512
KSIZE = 16
N_CHAN = 8
N_STEPS = 2
NUM_SUBCORES = 16
ROWS_P1 = N_NODES // NUM_SUBCORES
ROWS_P2 = N_NODES // 32
N_GROUPS = N_NODES // 4


def _build_tables(x_v, tab_v):
    def body(th, carry):
        for u in range(2):
            g = 2 * th + u
            xs = [x_v[4 * g + i, :] for i in range(4)]
            tab_v[pl.ds(g * 256, 16)] = jnp.zeros((KSIZE,), jnp.float32)
            vals = {}
            for s in range(1, 16):
                kk = (s & -s).bit_length() - 1
                prev = s ^ (1 << kk)
                vals[s] = xs[kk] if prev == 0 else vals[prev] + xs[kk]
                tab_v[pl.ds(g * 256 + s * 16, 16)] = vals[s]
        return carry

    lax.fori_loop(0, N_GROUPS // 2, body, 0)


def _masked_rowsums(m_v, tab_v, row_off, nrows):
    iota = lax.iota(jnp.int32, 16)
    sh4 = iota & 3
    fold1 = iota ^ 1
    fold2 = iota ^ 2
    rows = []

    for r0 in range(0, nrows, 4):

        def body(t, accs, r0=r0):
            out = []
            for ri in range(4):
                r = row_off + r0 + ri
                mrow = jnp.minimum(m_v[r, pl.ds(t * 16, 16)], 1)
                sh = mrow << sh4
                s1 = sh + sh.at[fold1].get(mode="promise_in_bounds")
                nib = s1 + s1.at[fold2].get(mode="promise_in_bounds")
                acc = accs[ri]
                for k in range(4):
                    nb = nib.at[jnp.full((16,), 4 * k, jnp.int32)].get(
                        mode="promise_in_bounds")
                    vidx = (nb << 4) + ((4 * t + k) << 8) + iota
                    acc = acc + plsc.load_gather(tab_v, [vidx])
                out.append(acc)
            return tuple(out)

        zero = jnp.zeros((KSIZE,), jnp.float32)
        accs = lax.fori_loop(0, N_NODES // 16, body,
                             tuple(zero for _ in range(4)))
        rows.extend(accs)
    return rows


@functools.cache
def _build_call():
    mesh = plsc.VectorSubcoreMesh(core_axis_name="c", subcore_axis_name="s")

    @functools.partial(
        pl.kernel,
        out_type=jax.ShapeDtypeStruct((N_CHAN, N_NODES, KSIZE), jnp.float32),
        mesh=mesh,
        compiler_params=pltpu.CompilerParams(
            use_tc_tiling_on_sc=False, needs_layout_passes=False),
        scratch_types=[
            pltpu.VMEM((ROWS_P1, N_NODES), jnp.int32),
            pltpu.VMEM((N_NODES, KSIZE), jnp.float32),
            pltpu.VMEM((N_NODES, KSIZE), jnp.float32),
            pltpu.VMEM((N_GROUPS * 256,), jnp.float32),
            pltpu.VMEM((N_CHAN, N_STEPS, KSIZE), jnp.float32),
            pltpu.VMEM((ROWS_P1, KSIZE), jnp.float32),
            pltpu.VMEM((N_CHAN, ROWS_P2, KSIZE), jnp.float32),
            pltpu.VMEM_SHARED((N_NODES, KSIZE), jnp.float32),
            pltpu.SemaphoreType.DMA,
            pltpu.SemaphoreType.DMA,
            pltpu.SemaphoreType.DMA,
        ],
    )
    def wl_conv(m_hbm, l_hbm, k_hbm, out_hbm,
                m_v, x_v, p_v, tab_v, k_v, o1_v, o_v, sh_p,
                sem_a, sem_b, sem_c):
        sid = lax.axis_index("s")
        cid = lax.axis_index("c")
        base1 = sid * ROWS_P1
        cp_a = pltpu.async_copy(m_hbm.at[pl.ds(base1, ROWS_P1), :], m_v,
                                sem_a)
        cp_b = pltpu.async_copy(l_hbm, x_v, sem_b)
        cp_c = pltpu.async_copy(k_hbm, k_v, sem_c)
        cp_b.wait()
        _build_tables(x_v, tab_v)
        cp_a.wait()

        rows = _masked_rowsums(m_v, tab_v, 0, ROWS_P1)
        for r in range(ROWS_P1):
            o1_v[r, :] = rows[r]
        pltpu.sync_copy(o1_v, sh_p.at[pl.ds(base1, ROWS_P1), :])
        plsc.subcore_barrier()
        pltpu.sync_copy(sh_p, p_v)

        _build_tables(p_v, tab_v)
        row_off = cid * ROWS_P2
        base2 = base1 + row_off
        qs = _masked_rowsums(m_v, tab_v, row_off, ROWS_P2)
        cp_c.wait()
        for r in range(ROWS_P2):
            q = qs[r]
            p_i = p_v[base2 + r, :]
            l_i = x_v[base2 + r, :]
            for c in range(N_CHAN):
                k0 = k_v[c, 0, :]
                k1 = k_v[c, 1, :]
                o_v[c, r, :] = l_i + p_i * (k0 + k1) + q * (k0 * k1)
        for c in range(N_CHAN):
            pltpu.sync_copy(o_v.at[c],
                            out_hbm.at[c].at[pl.ds(base2, ROWS_P2), :])

    return wl_conv


def kernel(labelsList, ligand_structure, kernels):
    wl_conv = _build_call()
    return wl_conv(ligand_structure, labelsList, kernels)
